# Initial kernel scaffold; baseline (speedup 1.0000x reference)
import jax
import jax.numpy as jnp
from jax.experimental import pallas as pl


def kernel(X, tables, alpha):
    raise NotImplementedError("write your pallas kernel here")


# speedup vs baseline: 1.0020x; 1.0020x over previous
import functools

import jax
import jax.numpy as jnp
from jax import lax
from jax.experimental import pallas as pl
from jax.experimental.pallas import tpu as pltpu
from jax.experimental.pallas import tpu_sc as plsc

_NF = 26
_V = 1000000
_B = 16384
_NW = 32
_BPW = _B // _NW
_L = 16
_CH = 128
_NCH = _BPW // _CH
_NC = _BPW // _L


def _tanh16(x):
    e = jnp.exp(2.0 * x)
    return (e - 1.0) / (e + 1.0)


_mesh = plsc.VectorSubcoreMesh(core_axis_name="c", subcore_axis_name="s")


@functools.partial(
    pl.kernel,
    mesh=_mesh,
    out_type=jax.ShapeDtypeStruct((_B,), jnp.float32),
    scratch_types=[
        pltpu.VMEM((_NF, _L), jnp.float32),
        pltpu.VMEM((_BPW,), jnp.int32),
        pltpu.VMEM((_BPW,), jnp.float32),
        pltpu.VMEM((_BPW,), jnp.float32),
        pltpu.SemaphoreType.DMA,
    ],
)
def _sc_linear(xt_hbm, flat_hbm, a_hbm, out_hbm, wv, iv, vv, acc, sem):
    cid = lax.axis_index("c")
    sid = lax.axis_index("s")
    wid = sid * 2 + cid
    base = wid * _BPW

    pltpu.sync_copy(a_hbm, wv)

    zero = jnp.zeros((_L,), jnp.float32)

    def _zero_body(c, _):
        acc[pl.ds(c * _L, _L)] = zero
        return 0

    lax.fori_loop(0, _NC, _zero_body, 0)

    lane = lax.iota(jnp.int32, _L)

    for f in range(_NF):
        pltpu.sync_copy(xt_hbm.at[f, wid], iv)

        off = jnp.full((_L,), f * _V, jnp.int32)

        def _idx_body(c, _):
            s = pl.ds(c * _L, _L)
            iv[s] = iv[s] + off
            return 0

        lax.fori_loop(0, _NC, _idx_body, 0)

        cps = []
        for j in range(_NCH):
            s = pl.ds(j * _CH, _CH)
            cps.append(pltpu.async_copy(flat_hbm.at[iv.at[s]], vv.at[s], sem))
        for cp in cps:
            cp.wait()

        wf = _tanh16(wv[f])

        def _acc_body(c, _):
            s = pl.ds(c * _L, _L)
            acc[s] = acc[s] + vv[s] * wf
            return 0

        lax.fori_loop(0, _NC, _acc_body, 0)

    pltpu.sync_copy(acc, out_hbm.at[pl.ds(base, _BPW)])


def kernel(X, tables, alpha):
    xt = X.T.reshape(_NF, _NW, _BPW)
    flat = tables.reshape(_NF * _V)
    a_b = jnp.broadcast_to(alpha[:, None], (_NF, _L))
    out = _sc_linear(xt, flat, a_b)
    return out.reshape(_B, 1)


# speedup vs baseline: 1.0145x; 1.0124x over previous
import functools

import jax
import jax.numpy as jnp
from jax import lax
from jax.experimental import pallas as pl
from jax.experimental.pallas import tpu as pltpu
from jax.experimental.pallas import tpu_sc as plsc

_NF = 26
_V = 1000000
_B = 16384
_NW = 32
_BPW = _B // _NW
_L = 16
_CH = 128
_NG = _NF * _BPW // _CH
_NC = _BPW // _L
_NCT = _NF * _NC


def _tanh16(x):
    e = jnp.exp(2.0 * x)
    return (e - 1.0) / (e + 1.0)


_mesh = plsc.VectorSubcoreMesh(core_axis_name="c", subcore_axis_name="s")


@functools.partial(
    pl.kernel,
    mesh=_mesh,
    out_type=jax.ShapeDtypeStruct((_B,), jnp.float32),
    scratch_types=[
        pltpu.VMEM((_NF, _L), jnp.float32),
        pltpu.VMEM((_NF * _BPW,), jnp.int32),
        pltpu.VMEM((_NF * _BPW,), jnp.float32),
        pltpu.VMEM((_BPW,), jnp.float32),
        pltpu.SemaphoreType.DMA,
        pltpu.SemaphoreType.DMA,
    ],
)
def _sc_linear(xt_hbm, flat_hbm, a_hbm, out_hbm, wv, iv, vv, acc, sem, gsem):
    cid = lax.axis_index("c")
    sid = lax.axis_index("s")
    wid = sid * 2 + cid
    base = wid * _BPW

    cps = []
    for f in range(_NF):
        cps.append(
            pltpu.async_copy(xt_hbm.at[f, wid], iv.at[pl.ds(f * _BPW, _BPW)], sem)
        )
    pltpu.sync_copy(a_hbm, wv)

    for f in range(_NF):
        wv[f] = _tanh16(wv[f])

    for cp in cps:
        cp.wait()

    def _idx_body(c, _):
        f = c // _NC
        s = pl.ds(c * _L, _L)
        iv[s] = iv[s] + jnp.broadcast_to(f * _V, (_L,))
        return 0

    lax.fori_loop(0, _NCT, _idx_body, 0)

    def _gather_body(j, _):
        s = pl.ds(j * _CH, _CH)
        pltpu.async_copy(flat_hbm.at[iv.at[s]], vv.at[s], gsem)
        return 0

    lax.fori_loop(0, _NG, _gather_body, 0)
    pltpu.make_async_copy(flat_hbm.at[pl.ds(0, _NF * _BPW)], vv, gsem).wait()

    zero = jnp.zeros((_L,), jnp.float32)

    def _zero_body(c, _):
        acc[pl.ds(c * _L, _L)] = zero
        return 0

    lax.fori_loop(0, _NC, _zero_body, 0)

    def _acc_body(c, _):
        f = c // _NC
        r = c % _NC
        s = pl.ds(r * _L, _L)
        acc[s] = acc[s] + vv[pl.ds(c * _L, _L)] * wv[f]
        return 0

    lax.fori_loop(0, _NCT, _acc_body, 0)

    pltpu.sync_copy(acc, out_hbm.at[pl.ds(base, _BPW)])


def kernel(X, tables, alpha):
    xt = X.T.reshape(_NF, _NW, _BPW)
    flat = tables.reshape(_NF * _V)
    a_b = jnp.broadcast_to(alpha[:, None], (_NF, _L))
    out = _sc_linear(xt, flat, a_b)
    return out.reshape(_B, 1)


# speedup vs baseline: 1.0150x; 1.0005x over previous
import functools

import jax
import jax.numpy as jnp
from jax import lax
from jax.experimental import pallas as pl
from jax.experimental.pallas import tpu as pltpu
from jax.experimental.pallas import tpu_sc as plsc

_NF = 26
_V = 1000000
_B = 16384
_NW = 32
_BPW = _B // _NW
_L = 16
_CH = 128
_NCH = _BPW // _CH
_NC = _BPW // _L


def _tanh16(x):
    e = jnp.exp(2.0 * x)
    return (e - 1.0) / (e + 1.0)


_mesh = plsc.VectorSubcoreMesh(core_axis_name="c", subcore_axis_name="s")


@functools.partial(
    pl.kernel,
    mesh=_mesh,
    out_type=jax.ShapeDtypeStruct((_B,), jnp.float32),
    scratch_types=[
        pltpu.VMEM((_NF, _L), jnp.float32),
        pltpu.VMEM((_NF * _BPW,), jnp.int32),
        pltpu.VMEM((_NF * _BPW,), jnp.float32),
        pltpu.VMEM((_BPW,), jnp.float32),
        pltpu.SemaphoreType.DMA,
        pltpu.SemaphoreType.DMA,
        pltpu.SemaphoreType.DMA,
    ],
)
def _sc_linear(xt_hbm, flat_hbm, a_hbm, out_hbm, wv, iv, vv, acc, sem, g0, g1):
    cid = lax.axis_index("c")
    sid = lax.axis_index("s")
    wid = sid * 2 + cid
    base = wid * _BPW
    gsems = (g0, g1)

    stage = []
    for f in range(_NF):
        stage.append(
            pltpu.async_copy(xt_hbm.at[f, wid], iv.at[pl.ds(f * _BPW, _BPW)], sem)
        )
    pltpu.sync_copy(a_hbm, wv)
    for f in range(_NF):
        wv[f] = _tanh16(wv[f])

    def _build(f):
        off = jnp.broadcast_to(f * _V, (_L,))

        def body(c, _):
            s = pl.ds(f * _BPW + c * _L, _L)
            iv[s] = iv[s] + off
            return 0

        lax.fori_loop(0, _NC, body, 0)

    def _fire(f):
        cps = []
        for j in range(_NCH):
            s = pl.ds(f * _BPW + j * _CH, _CH)
            cps.append(pltpu.async_copy(flat_hbm.at[iv.at[s]], vv.at[s], gsems[f % 2]))
        return cps

    def _accum(f):
        wf = wv[f]
        if f == 0:
            def body0(c, _):
                s = pl.ds(c * _L, _L)
                acc[s] = vv[s] * wf
                return 0

            lax.fori_loop(0, _NC, body0, 0)
        else:
            def body(c, _):
                s = pl.ds(c * _L, _L)
                acc[s] = acc[s] + vv[pl.ds(f * _BPW + c * _L, _L)] * wf
                return 0

            lax.fori_loop(0, _NC, body, 0)

    gcps = None
    for f in range(_NF):
        stage[f].wait()
        _build(f)
        nxt = _fire(f)
        if gcps is not None:
            for cp in gcps:
                cp.wait()
            _accum(f - 1)
        gcps = nxt
    for cp in gcps:
        cp.wait()
    _accum(_NF - 1)

    pltpu.sync_copy(acc, out_hbm.at[pl.ds(base, _BPW)])


def kernel(X, tables, alpha):
    xt = X.T.reshape(_NF, _NW, _BPW)
    flat = tables.reshape(_NF * _V)
    a_b = jnp.broadcast_to(alpha[:, None], (_NF, _L))
    out = _sc_linear(xt, flat, a_b)
    return out.reshape(_B, 1)


# speedup vs baseline: 5.1100x; 5.0346x over previous
import functools

import jax
import jax.numpy as jnp
from jax import lax
from jax.experimental import pallas as pl
from jax.experimental.pallas import tpu as pltpu
from jax.experimental.pallas import tpu_sc as plsc

_NF = 26
_V = 1000000
_B = 16384
_NW = 32
_BPW = _B // _NW
_L = 16
_CH = 128
_NCH = _BPW // _CH
_NC = _BPW // _L


def _tanh16(x):
    e = jnp.exp(2.0 * x)
    return (e - 1.0) / (e + 1.0)


_mesh = plsc.VectorSubcoreMesh(core_axis_name="c", subcore_axis_name="s")


@functools.partial(
    pl.kernel,
    mesh=_mesh,
    compiler_params=pltpu.CompilerParams(use_tc_tiling_on_sc=False),
    out_type=jax.ShapeDtypeStruct((_B,), jnp.float32),
    scratch_types=[
        pltpu.VMEM((_NF, _L), jnp.float32),
        pltpu.VMEM((_NF * _BPW,), jnp.int32),
        pltpu.VMEM((_NF * _BPW,), jnp.float32),
        pltpu.VMEM((_BPW,), jnp.float32),
        pltpu.SemaphoreType.DMA,
        pltpu.SemaphoreType.DMA,
        pltpu.SemaphoreType.DMA,
    ],
)
def _sc_linear(xt_hbm, *rest):
    tabs = rest[:_NF]
    a_hbm = rest[_NF]
    out_hbm = rest[_NF + 1]
    wv, iv, vv, acc, sem, g0, g1 = rest[_NF + 2:]
    cid = lax.axis_index("c")
    sid = lax.axis_index("s")
    wid = sid * 2 + cid
    base = wid * _BPW
    gsems = (g0, g1)

    stage = []
    for f in range(_NF):
        stage.append(
            pltpu.async_copy(xt_hbm.at[f, wid], iv.at[pl.ds(f * _BPW, _BPW)], sem)
        )
    pltpu.sync_copy(a_hbm, wv)
    for f in range(_NF):
        wv[f] = _tanh16(wv[f])

    def _fire(f):
        cps = []
        for j in range(_NCH):
            s = pl.ds(f * _BPW + j * _CH, _CH)
            cps.append(pltpu.async_copy(tabs[f].at[iv.at[s]], vv.at[s], gsems[f % 2]))
        return cps

    def _accum(f):
        wf = wv[f]
        if f == 0:
            def body0(c, _):
                s = pl.ds(c * _L, _L)
                acc[s] = vv[s] * wf
                return 0

            lax.fori_loop(0, _NC, body0, 0)
        else:
            def body(c, _):
                s = pl.ds(c * _L, _L)
                acc[s] = acc[s] + vv[pl.ds(f * _BPW + c * _L, _L)] * wf
                return 0

            lax.fori_loop(0, _NC, body, 0)

    gcps = None
    for f in range(_NF):
        stage[f].wait()
        nxt = _fire(f)
        if gcps is not None:
            for cp in gcps:
                cp.wait()
            _accum(f - 1)
        gcps = nxt
    for cp in gcps:
        cp.wait()
    _accum(_NF - 1)

    pltpu.sync_copy(acc, out_hbm.at[pl.ds(base, _BPW)])


def kernel(X, tables, alpha):
    xt = X.T.reshape(_NF, _NW, _BPW)
    a_b = jnp.broadcast_to(alpha[:, None], (_NF, _L))
    tabs = [tables[f, :, 0] for f in range(_NF)]
    out = _sc_linear(xt, *tabs, a_b)
    return out.reshape(_B, 1)
